# Initial kernel scaffold; baseline (speedup 1.0000x reference)
import jax
import jax.numpy as jnp
from jax.experimental import pallas as pl


def kernel(features, positions, batch, W_enc, b_enc, Wq, bq, Wk, bk, Wv, bv, Wp1, bp1, Wp2, bp2, Wa1, ba1, Wa2, ba2, W_dec, b_dec):
    raise NotImplementedError("write your pallas kernel here")


# speedup vs baseline: 17.7804x; 17.7804x over previous
import functools

import jax
import jax.numpy as jnp
from jax import lax
from jax.experimental import pallas as pl
from jax.experimental.pallas import tpu as pltpu
from jax.experimental.pallas import tpu_sc as plsc

KNN = 16
QB = 256
QC = 256
DTAB = 128


def _mm(a, w):
    return jnp.dot(a.astype(jnp.bfloat16), w.astype(jnp.bfloat16),
                   preferred_element_type=jnp.float32)


def _knn_encode_body(posT_ref, pos_ref, feat_ref,
                     Wenc_ref, benc_ref, Wq_ref, bq_ref, Wk_ref, bk_ref,
                     Wv_ref, bv_ref,
                     nbr_ref, q_ref, tab_ref, *, n_points):
    b = pl.program_id(0)
    n = n_points

    posT = posT_ref[0]
    xa = posT[0:1, :]
    ya = posT[1:2, :]
    za = posT[2:3, :]
    sqa = (xa * xa + ya * ya) + za * za

    p = pos_ref[...]
    xq = p[:, 0:1]
    yq = p[:, 1:2]
    zq = p[:, 2:3]
    sqq = (xq * xq + yq * yq) + zq * zq

    r = lambda t: t.astype(jnp.bfloat16).astype(jnp.float32)
    dot = (r(xq) * r(xa) + r(yq) * r(ya)) + r(zq) * r(za)
    d = (sqq + sqa) - 2.0 * dot

    iota = lax.broadcasted_iota(jnp.int32, (QB, n), 1)
    cols = []
    for _ in range(KNN):
        m = jnp.min(d, axis=1, keepdims=True)
        hit = d == m
        idx = jnp.min(jnp.where(hit, iota, n), axis=1, keepdims=True)
        cols.append(idx)
        d = jnp.where(iota == idx, jnp.inf, d)
    nbr = jnp.concatenate(cols, axis=1) + b * n
    nbr_ref[...] = nbr

    f = feat_ref[...]
    x = _mm(f, Wenc_ref[...]) + benc_ref[...]
    q = _mm(x, Wq_ref[...]) + bq_ref[...]
    k = _mm(x, Wk_ref[...]) + bk_ref[...]
    v = _mm(x, Wv_ref[...]) + bv_ref[...]
    q_ref[...] = q
    pad = jnp.zeros((QB, DTAB - 64 - 3), jnp.float32)
    tab_ref[...] = jnp.concatenate([k, v, p, pad], axis=1)


def _attn_body(q_ref, g_ref, pos_ref, feat_ref,
               Wp1_ref, bp1_ref, Wp2_ref, bp2_ref,
               Wa1_ref, ba1_ref, Wa2_ref, ba2_ref,
               Wdec_ref, bdec_ref, out_ref):
    C = q_ref.shape[1]
    g = g_ref[...]
    g3 = g.reshape(QC, KNN, DTAB)
    k_n = g3[:, :, 0:C]
    v_n = g3[:, :, C:2 * C]
    pos = pos_ref[...]

    rx = pos[:, None, 0:1] - g3[:, :, 2 * C:2 * C + 1]
    ry = pos[:, None, 1:2] - g3[:, :, 2 * C + 1:2 * C + 2]
    rz = pos[:, None, 2:3] - g3[:, :, 2 * C + 2:2 * C + 3]

    r = lambda t: t.astype(jnp.bfloat16).astype(jnp.float32)
    w0 = r(Wp1_ref[0:1, :])
    w1 = r(Wp1_ref[1:2, :])
    w2 = r(Wp1_ref[2:3, :])
    h = (r(rx) * w0 + r(ry) * w1) + r(rz) * w2 + bp1_ref[...]
    h = jnp.maximum(h, 0.0)
    h2 = h.reshape(QC * KNN, C)
    delta = _mm(h2, Wp2_ref[...]) + bp2_ref[...]
    delta3 = delta.reshape(QC, KNN, C)

    q = q_ref[...]
    e3 = (q[:, None, :] - k_n) + delta3
    e2 = e3.reshape(QC * KNN, C)
    a = jnp.maximum(_mm(e2, Wa1_ref[...]) + ba1_ref[...], 0.0)
    gamma = _mm(a, Wa2_ref[...]) + ba2_ref[...]
    g3m = gamma.reshape(QC, KNN, C)

    mx = jnp.max(g3m, axis=1, keepdims=True)
    ex = jnp.exp(g3m - mx)
    sm = jnp.sum(ex, axis=1, keepdims=True)
    alpha = ex / sm
    out = jnp.sum(alpha * (v_n + delta3), axis=1)

    dec = _mm(out, Wdec_ref[...]) + bdec_ref[...]
    out_ref[...] = feat_ref[...] + dec


def _sc_gather(table, idx_flat):
    rows = idx_flat.shape[0]
    info = plsc.get_sparse_core_info()
    nw = info.num_cores * info.num_subcores
    per_w = rows // nw
    chunk = 128
    n_iter = per_w // chunk
    mesh = plsc.VectorSubcoreMesh(core_axis_name="c", subcore_axis_name="s")

    @functools.partial(
        pl.kernel, mesh=mesh,
        out_type=jax.ShapeDtypeStruct((rows, DTAB), jnp.float32),
        scratch_types=[
            pltpu.VMEM((chunk,), jnp.int32),
            pltpu.VMEM((chunk, DTAB), jnp.float32),
            pltpu.SemaphoreType.DMA,
        ],
    )
    def gather_k(tab_hbm, idx_hbm, out_hbm, idx_v, rows_v, sem):
        wid = lax.axis_index("s") * info.num_cores + lax.axis_index("c")
        base = wid * per_w

        def body(i, carry):
            off = base + i * chunk
            pltpu.sync_copy(idx_hbm.at[pl.ds(off, chunk)], idx_v)
            pltpu.async_copy(tab_hbm.at[idx_v], rows_v, sem).wait()
            pltpu.sync_copy(rows_v, out_hbm.at[pl.ds(off, chunk)])
            return carry

        lax.fori_loop(0, n_iter, body, 0)

    return gather_k(table, idx_flat)


def kernel(features, positions, batch, W_enc, b_enc, Wq, bq, Wk, bk, Wv, bv,
           Wp1, bp1, Wp2, bp2, Wa1, ba1, Wa2, ba2, W_dec, b_dec):
    B, N, F = features.shape
    C = Wq.shape[0]
    nbq = N // QB

    f2 = features.reshape(B * N, F)
    pos2 = positions.reshape(B * N, 3)
    posT = jnp.transpose(positions, (0, 2, 1))
    b_enc2 = b_enc.reshape(1, C)
    bq2 = bq.reshape(1, C)
    bk2 = bk.reshape(1, C)
    bv2 = bv.reshape(1, C)

    row_spec = lambda w: pl.BlockSpec((QB, w), lambda b, i: (b * nbq + i, 0))
    full = lambda shape: pl.BlockSpec(shape, lambda b, i: tuple(0 for _ in shape))

    nbr, q, tab = pl.pallas_call(
        functools.partial(_knn_encode_body, n_points=N),
        grid=(B, nbq),
        in_specs=[
            pl.BlockSpec((1, 3, N), lambda b, i: (b, 0, 0)),
            row_spec(3),
            row_spec(F),
            full((F, C)), full((1, C)),
            full((C, C)), full((1, C)),
            full((C, C)), full((1, C)),
            full((C, C)), full((1, C)),
        ],
        out_specs=[row_spec(KNN), row_spec(C), row_spec(DTAB)],
        out_shape=[
            jax.ShapeDtypeStruct((B * N, KNN), jnp.int32),
            jax.ShapeDtypeStruct((B * N, C), jnp.float32),
            jax.ShapeDtypeStruct((B * N, DTAB), jnp.float32),
        ],
    )(posT, pos2, f2, W_enc, b_enc2, Wq, bq2, Wk, bk2, Wv, bv2)

    gathered = _sc_gather(tab, nbr.reshape(-1))

    nqc = (B * N) // QC
    rs = lambda w: pl.BlockSpec((QC, w), lambda i: (i, 0))
    fullc = lambda shape: pl.BlockSpec(shape, lambda i: tuple(0 for _ in shape))
    featout = pl.pallas_call(
        _attn_body,
        grid=(nqc,),
        in_specs=[
            rs(C),
            pl.BlockSpec((QC * KNN, DTAB), lambda i: (i, 0)),
            rs(3),
            rs(F),
            fullc((3, C)), fullc((1, C)),
            fullc((C, C)), fullc((1, C)),
            fullc((C, C)), fullc((1, C)),
            fullc((C, C)), fullc((1, C)),
            fullc((C, F)), fullc((1, F)),
        ],
        out_specs=rs(F),
        out_shape=jax.ShapeDtypeStruct((B * N, F), jnp.float32),
    )(q, gathered, pos2, f2,
      Wp1, bp1.reshape(1, C), Wp2, bp2.reshape(1, C),
      Wa1, ba1.reshape(1, C), Wa2, ba2.reshape(1, C),
      W_dec, b_dec.reshape(1, F))

    return (featout.reshape(B, N, F), positions, batch)


# speedup vs baseline: 20.8762x; 1.1741x over previous
import functools

import jax
import jax.numpy as jnp
from jax import lax
from jax.experimental import pallas as pl
from jax.experimental.pallas import tpu as pltpu
from jax.experimental.pallas import tpu_sc as plsc

KNN = 16
QB = 256
QC = 256
DTAB = 128


def _mm(a, w):
    return jnp.dot(a.astype(jnp.bfloat16), w.astype(jnp.bfloat16),
                   preferred_element_type=jnp.float32)


def _knn_encode_body(posT_ref, pos_ref, feat_ref,
                     Wenc_ref, benc_ref, Wq_ref, bq_ref, Wk_ref, bk_ref,
                     Wv_ref, bv_ref,
                     nbr_ref, q_ref, tab_ref, *, n_points):
    b = pl.program_id(0)
    n = n_points

    posT = posT_ref[0]
    xa = posT[0:1, :]
    ya = posT[1:2, :]
    za = posT[2:3, :]
    sqa = (xa * xa + ya * ya) + za * za

    p = pos_ref[...]
    xq = p[:, 0:1]
    yq = p[:, 1:2]
    zq = p[:, 2:3]
    sqq = (xq * xq + yq * yq) + zq * zq

    r = lambda t: t.astype(jnp.bfloat16).astype(jnp.float32)
    dot = (r(xq) * r(xa) + r(yq) * r(ya)) + r(zq) * r(za)
    d = (sqq + sqa) - 2.0 * dot

    iotaf = lax.broadcasted_iota(jnp.int32, (QB, n), 1).astype(jnp.float32)
    fbig = jnp.float32(float(n))
    cols = []
    for _ in range(KNN):
        m = jnp.min(d, axis=1, keepdims=True)
        pick = jnp.where(d == m, iotaf, fbig)
        idxf = jnp.min(pick, axis=1, keepdims=True)
        cols.append(idxf)
        d = jnp.where(iotaf == idxf, jnp.inf, d)
    nbr = jnp.concatenate(cols, axis=1).astype(jnp.int32) + b * n
    nbr_ref[...] = nbr

    f = feat_ref[...]
    x = _mm(f, Wenc_ref[...]) + benc_ref[...]
    q = _mm(x, Wq_ref[...]) + bq_ref[...]
    k = _mm(x, Wk_ref[...]) + bk_ref[...]
    v = _mm(x, Wv_ref[...]) + bv_ref[...]
    q_ref[...] = q
    pad = jnp.zeros((QB, DTAB - 64 - 3), jnp.float32)
    tab_ref[...] = jnp.concatenate([k, v, p, pad], axis=1)


def _attn_body(q_ref, g_ref, pos_ref, feat_ref,
               Wp1_ref, bp1_ref, Wp2_ref, bp2_ref,
               Wa1_ref, ba1_ref, Wa2_ref, ba2_ref,
               Wdec_ref, bdec_ref, out_ref):
    C = q_ref.shape[1]
    g = g_ref[...]
    g3 = g.reshape(QC, KNN, DTAB)
    k_n = g3[:, :, 0:C]
    v_n = g3[:, :, C:2 * C]
    pos = pos_ref[...]

    rx = pos[:, None, 0:1] - g3[:, :, 2 * C:2 * C + 1]
    ry = pos[:, None, 1:2] - g3[:, :, 2 * C + 1:2 * C + 2]
    rz = pos[:, None, 2:3] - g3[:, :, 2 * C + 2:2 * C + 3]

    r = lambda t: t.astype(jnp.bfloat16).astype(jnp.float32)
    w0 = r(Wp1_ref[0:1, :])
    w1 = r(Wp1_ref[1:2, :])
    w2 = r(Wp1_ref[2:3, :])
    h = (r(rx) * w0 + r(ry) * w1) + r(rz) * w2 + bp1_ref[...]
    h = jnp.maximum(h, 0.0)
    h2 = h.reshape(QC * KNN, C)
    delta = _mm(h2, Wp2_ref[...]) + bp2_ref[...]
    delta3 = delta.reshape(QC, KNN, C)

    q = q_ref[...]
    e3 = (q[:, None, :] - k_n) + delta3
    e2 = e3.reshape(QC * KNN, C)
    a = jnp.maximum(_mm(e2, Wa1_ref[...]) + ba1_ref[...], 0.0)
    gamma = _mm(a, Wa2_ref[...]) + ba2_ref[...]
    g3m = gamma.reshape(QC, KNN, C)

    mx = jnp.max(g3m, axis=1, keepdims=True)
    ex = jnp.exp(g3m - mx)
    sm = jnp.sum(ex, axis=1, keepdims=True)
    alpha = ex / sm
    out = jnp.sum(alpha * (v_n + delta3), axis=1)

    dec = _mm(out, Wdec_ref[...]) + bdec_ref[...]
    out_ref[...] = feat_ref[...] + dec


def _sc_gather(table, idx_flat):
    rows = idx_flat.shape[0]
    info = plsc.get_sparse_core_info()
    nw = info.num_cores * info.num_subcores
    per_w = rows // nw
    chunk = 128
    n_iter = per_w // chunk
    mesh = plsc.VectorSubcoreMesh(core_axis_name="c", subcore_axis_name="s")

    @functools.partial(
        pl.kernel, mesh=mesh,
        out_type=jax.ShapeDtypeStruct((rows, DTAB), jnp.float32),
        scratch_types=[
            pltpu.VMEM((chunk,), jnp.int32),
            pltpu.VMEM((chunk, DTAB), jnp.float32),
            pltpu.SemaphoreType.DMA,
        ],
    )
    def gather_k(tab_hbm, idx_hbm, out_hbm, idx_v, rows_v, sem):
        wid = lax.axis_index("s") * info.num_cores + lax.axis_index("c")
        base = wid * per_w

        def body(i, carry):
            off = base + i * chunk
            pltpu.sync_copy(idx_hbm.at[pl.ds(off, chunk)], idx_v)
            pltpu.async_copy(tab_hbm.at[idx_v], rows_v, sem).wait()
            pltpu.sync_copy(rows_v, out_hbm.at[pl.ds(off, chunk)])
            return carry

        lax.fori_loop(0, n_iter, body, 0)

    return gather_k(table, idx_flat)


def kernel(features, positions, batch, W_enc, b_enc, Wq, bq, Wk, bk, Wv, bv,
           Wp1, bp1, Wp2, bp2, Wa1, ba1, Wa2, ba2, W_dec, b_dec):
    B, N, F = features.shape
    C = Wq.shape[0]
    nbq = N // QB

    f2 = features.reshape(B * N, F)
    pos2 = positions.reshape(B * N, 3)
    posT = jnp.transpose(positions, (0, 2, 1))
    b_enc2 = b_enc.reshape(1, C)
    bq2 = bq.reshape(1, C)
    bk2 = bk.reshape(1, C)
    bv2 = bv.reshape(1, C)

    row_spec = lambda w: pl.BlockSpec((QB, w), lambda b, i: (b * nbq + i, 0))
    full = lambda shape: pl.BlockSpec(shape, lambda b, i: tuple(0 for _ in shape))

    nbr, q, tab = pl.pallas_call(
        functools.partial(_knn_encode_body, n_points=N),
        grid=(B, nbq),
        in_specs=[
            pl.BlockSpec((1, 3, N), lambda b, i: (b, 0, 0)),
            row_spec(3),
            row_spec(F),
            full((F, C)), full((1, C)),
            full((C, C)), full((1, C)),
            full((C, C)), full((1, C)),
            full((C, C)), full((1, C)),
        ],
        out_specs=[row_spec(KNN), row_spec(C), row_spec(DTAB)],
        out_shape=[
            jax.ShapeDtypeStruct((B * N, KNN), jnp.int32),
            jax.ShapeDtypeStruct((B * N, C), jnp.float32),
            jax.ShapeDtypeStruct((B * N, DTAB), jnp.float32),
        ],
    )(posT, pos2, f2, W_enc, b_enc2, Wq, bq2, Wk, bk2, Wv, bv2)

    gathered = _sc_gather(tab, nbr.reshape(-1))

    nqc = (B * N) // QC
    rs = lambda w: pl.BlockSpec((QC, w), lambda i: (i, 0))
    fullc = lambda shape: pl.BlockSpec(shape, lambda i: tuple(0 for _ in shape))
    featout = pl.pallas_call(
        _attn_body,
        grid=(nqc,),
        in_specs=[
            rs(C),
            pl.BlockSpec((QC * KNN, DTAB), lambda i: (i, 0)),
            rs(3),
            rs(F),
            fullc((3, C)), fullc((1, C)),
            fullc((C, C)), fullc((1, C)),
            fullc((C, C)), fullc((1, C)),
            fullc((C, C)), fullc((1, C)),
            fullc((C, F)), fullc((1, F)),
        ],
        out_specs=rs(F),
        out_shape=jax.ShapeDtypeStruct((B * N, F), jnp.float32),
    )(q, gathered, pos2, f2,
      Wp1, bp1.reshape(1, C), Wp2, bp2.reshape(1, C),
      Wa1, ba1.reshape(1, C), Wa2, ba2.reshape(1, C),
      W_dec, b_dec.reshape(1, F))

    return (featout.reshape(B, N, F), positions, batch)


# speedup vs baseline: 23.4345x; 1.1225x over previous
import functools

import jax
import jax.numpy as jnp
from jax import lax
from jax.experimental import pallas as pl
from jax.experimental.pallas import tpu as pltpu
from jax.experimental.pallas import tpu_sc as plsc

KNN = 16
QB = 256
QC = 256
DTAB = 128


def _mm(a, w):
    return jnp.dot(a.astype(jnp.bfloat16), w.astype(jnp.bfloat16),
                   preferred_element_type=jnp.float32)


def _knn_encode_body(posT_ref, pos_ref, feat_ref,
                     Wenc_ref, benc_ref, Wq_ref, bq_ref, Wk_ref, bk_ref,
                     Wv_ref, bv_ref,
                     nbr_ref, q_ref, tab_ref, *, n_points):
    b = pl.program_id(0)
    n = n_points

    posT = posT_ref[0]
    xa = posT[0:1, :]
    ya = posT[1:2, :]
    za = posT[2:3, :]
    sqa = (xa * xa + ya * ya) + za * za

    p = pos_ref[...]
    xq = p[:, 0:1]
    yq = p[:, 1:2]
    zq = p[:, 2:3]
    sqq = (xq * xq + yq * yq) + zq * zq

    r = lambda t: t.astype(jnp.bfloat16).astype(jnp.float32)
    dot = (r(xq) * r(xa) + r(yq) * r(ya)) + r(zq) * r(za)
    d = (sqq + sqa) - 2.0 * dot

    iotaf = lax.broadcasted_iota(jnp.int32, (QB, n), 1).astype(jnp.float32)
    fbig = jnp.float32(float(n))
    cols = []
    for _ in range(KNN):
        m = jnp.min(d, axis=1, keepdims=True)
        pick = jnp.where(d == m, iotaf, fbig)
        idxf = jnp.min(pick, axis=1, keepdims=True)
        cols.append(idxf)
        d = jnp.where(iotaf == idxf, jnp.inf, d)
    nbr = jnp.concatenate(cols, axis=1).astype(jnp.int32) + b * n
    nbr_ref[...] = nbr

    f = feat_ref[...]
    x = _mm(f, Wenc_ref[...]) + benc_ref[...]
    q = _mm(x, Wq_ref[...]) + bq_ref[...]
    k = _mm(x, Wk_ref[...]) + bk_ref[...]
    v = _mm(x, Wv_ref[...]) + bv_ref[...]
    q_ref[...] = q
    pad = jnp.zeros((QB, DTAB - 64 - 3), jnp.float32)
    tab_ref[...] = jnp.concatenate([k, v, p, pad], axis=1)


def _attn_body(q_ref, g_ref, pos_ref, feat_ref,
               Wp1_ref, bp1_ref, Wp2_ref, bp2_ref,
               Wa1_ref, ba1_ref, Wa2_ref, ba2_ref,
               Wdec_ref, bdec_ref, out_ref):
    C = q_ref.shape[1]
    g = g_ref[...]
    g3 = g.reshape(QC, KNN, DTAB)
    k_n = g3[:, :, 0:C]
    v_n = g3[:, :, C:2 * C]
    pos = pos_ref[...]

    rx = pos[:, None, 0:1] - g3[:, :, 2 * C:2 * C + 1]
    ry = pos[:, None, 1:2] - g3[:, :, 2 * C + 1:2 * C + 2]
    rz = pos[:, None, 2:3] - g3[:, :, 2 * C + 2:2 * C + 3]

    r = lambda t: t.astype(jnp.bfloat16).astype(jnp.float32)
    w0 = r(Wp1_ref[0:1, :])
    w1 = r(Wp1_ref[1:2, :])
    w2 = r(Wp1_ref[2:3, :])
    h = (r(rx) * w0 + r(ry) * w1) + r(rz) * w2 + bp1_ref[...]
    h = jnp.maximum(h, 0.0)
    h2 = h.reshape(QC * KNN, C)
    delta = _mm(h2, Wp2_ref[...]) + bp2_ref[...]
    delta3 = delta.reshape(QC, KNN, C)

    q = q_ref[...]
    e3 = (q[:, None, :] - k_n) + delta3
    e2 = e3.reshape(QC * KNN, C)
    a = jnp.maximum(_mm(e2, Wa1_ref[...]) + ba1_ref[...], 0.0)
    gamma = _mm(a, Wa2_ref[...]) + ba2_ref[...]
    g3m = gamma.reshape(QC, KNN, C)

    mx = jnp.max(g3m, axis=1, keepdims=True)
    ex = jnp.exp(g3m - mx)
    sm = jnp.sum(ex, axis=1, keepdims=True)
    alpha = ex / sm
    out = jnp.sum(alpha * (v_n + delta3), axis=1)

    dec = _mm(out, Wdec_ref[...]) + bdec_ref[...]
    out_ref[...] = feat_ref[...] + dec


def _sc_gather(table, idx_flat):
    rows = idx_flat.shape[0]
    info = plsc.get_sparse_core_info()
    nw = info.num_cores * info.num_subcores
    per_w = rows // nw
    chunk = 128
    n_iter = per_w // chunk
    mesh = plsc.VectorSubcoreMesh(core_axis_name="c", subcore_axis_name="s")

    @functools.partial(
        pl.kernel, mesh=mesh,
        out_type=jax.ShapeDtypeStruct((rows, DTAB), jnp.float32),
        scratch_types=[
            pltpu.VMEM((chunk,), jnp.int32),
            pltpu.VMEM((chunk, DTAB), jnp.float32),
            pltpu.SemaphoreType.DMA,
        ],
    )
    def gather_k(tab_hbm, idx_hbm, out_hbm, idx_v, rows_v, sem):
        wid = lax.axis_index("s") * info.num_cores + lax.axis_index("c")
        base = wid * per_w

        def body(i, carry):
            off = base + i * chunk
            pltpu.sync_copy(idx_hbm.at[pl.ds(off, chunk)], idx_v)
            pltpu.async_copy(tab_hbm.at[idx_v], rows_v, sem).wait()
            pltpu.sync_copy(rows_v, out_hbm.at[pl.ds(off, chunk)])
            return carry

        lax.fori_loop(0, n_iter, body, 0)

    return gather_k(table, idx_flat)


def kernel(features, positions, batch, W_enc, b_enc, Wq, bq, Wk, bk, Wv, bv,
           Wp1, bp1, Wp2, bp2, Wa1, ba1, Wa2, ba2, W_dec, b_dec):
    B = features.shape[0]
    h = B // 2
    out0 = _half(features[:h], positions[:h], W_enc, b_enc, Wq, bq, Wk, bk,
                 Wv, bv, Wp1, bp1, Wp2, bp2, Wa1, ba1, Wa2, ba2, W_dec, b_dec)
    out1 = _half(features[h:], positions[h:], W_enc, b_enc, Wq, bq, Wk, bk,
                 Wv, bv, Wp1, bp1, Wp2, bp2, Wa1, ba1, Wa2, ba2, W_dec, b_dec)
    return (jnp.concatenate([out0, out1], axis=0), positions, batch)


def _half(features, positions, W_enc, b_enc, Wq, bq, Wk, bk, Wv, bv,
          Wp1, bp1, Wp2, bp2, Wa1, ba1, Wa2, ba2, W_dec, b_dec):
    B, N, F = features.shape
    C = Wq.shape[0]
    nbq = N // QB

    f2 = features.reshape(B * N, F)
    pos2 = positions.reshape(B * N, 3)
    posT = jnp.transpose(positions, (0, 2, 1))
    b_enc2 = b_enc.reshape(1, C)
    bq2 = bq.reshape(1, C)
    bk2 = bk.reshape(1, C)
    bv2 = bv.reshape(1, C)

    row_spec = lambda w: pl.BlockSpec((QB, w), lambda b, i: (b * nbq + i, 0))
    full = lambda shape: pl.BlockSpec(shape, lambda b, i: tuple(0 for _ in shape))

    nbr, q, tab = pl.pallas_call(
        functools.partial(_knn_encode_body, n_points=N),
        grid=(B, nbq),
        in_specs=[
            pl.BlockSpec((1, 3, N), lambda b, i: (b, 0, 0)),
            row_spec(3),
            row_spec(F),
            full((F, C)), full((1, C)),
            full((C, C)), full((1, C)),
            full((C, C)), full((1, C)),
            full((C, C)), full((1, C)),
        ],
        out_specs=[row_spec(KNN), row_spec(C), row_spec(DTAB)],
        out_shape=[
            jax.ShapeDtypeStruct((B * N, KNN), jnp.int32),
            jax.ShapeDtypeStruct((B * N, C), jnp.float32),
            jax.ShapeDtypeStruct((B * N, DTAB), jnp.float32),
        ],
    )(posT, pos2, f2, W_enc, b_enc2, Wq, bq2, Wk, bk2, Wv, bv2)

    gathered = _sc_gather(tab, nbr.reshape(-1))

    nqc = (B * N) // QC
    rs = lambda w: pl.BlockSpec((QC, w), lambda i: (i, 0))
    fullc = lambda shape: pl.BlockSpec(shape, lambda i: tuple(0 for _ in shape))
    featout = pl.pallas_call(
        _attn_body,
        grid=(nqc,),
        in_specs=[
            rs(C),
            pl.BlockSpec((QC * KNN, DTAB), lambda i: (i, 0)),
            rs(3),
            rs(F),
            fullc((3, C)), fullc((1, C)),
            fullc((C, C)), fullc((1, C)),
            fullc((C, C)), fullc((1, C)),
            fullc((C, C)), fullc((1, C)),
            fullc((C, F)), fullc((1, F)),
        ],
        out_specs=rs(F),
        out_shape=jax.ShapeDtypeStruct((B * N, F), jnp.float32),
    )(q, gathered, pos2, f2,
      Wp1, bp1.reshape(1, C), Wp2, bp2.reshape(1, C),
      Wa1, ba1.reshape(1, C), Wa2, ba2.reshape(1, C),
      W_dec, b_dec.reshape(1, F))

    return featout.reshape(B, N, F)


# speedup vs baseline: 26.4148x; 1.1272x over previous
import functools

import jax
import jax.numpy as jnp
from jax import lax
from jax.experimental import pallas as pl
from jax.experimental.pallas import tpu as pltpu
from jax.experimental.pallas import tpu_sc as plsc

KNN = 16
QB = 256
QC = 256
DTAB = 128


def _mm(a, w):
    return jnp.dot(a.astype(jnp.bfloat16), w.astype(jnp.bfloat16),
                   preferred_element_type=jnp.float32)


NCH = 32
CH = 128
ROUNDS = 6


def _knn_encode_body(posq_ref, posf_ref, feat_ref,
                     Wenc_ref, benc_ref, Wq_ref, bq_ref, Wk_ref, bk_ref,
                     Wv_ref, bv_ref,
                     nbrT_ref, q_ref, tab_ref, *, n_points):
    b = pl.program_id(0)
    n = n_points

    r = lambda t: t.astype(jnp.bfloat16).astype(jnp.float32)
    pq = posq_ref[0]
    xq = pq[0:1, :]
    yq = pq[1:2, :]
    zq = pq[2:3, :]
    sqq = (xq * xq + yq * yq) + zq * zq
    pf = posf_ref[...]
    xa = pf[:, 0:1]
    ya = pf[:, 1:2]
    za = pf[:, 2:3]
    sqa = (xa * xa + ya * ya) + za * za

    def build_d3():
        dot = (r(xa) * r(xq) + r(ya) * r(yq)) + r(za) * r(zq)
        dT = (sqq + sqa) - 2.0 * dot
        return dT.reshape(NCH, CH, QB)

    d3 = build_d3()
    iwf = lax.broadcasted_iota(jnp.int32, (NCH, CH, QB), 1).astype(jnp.float32)
    fbig = jnp.float32(1e9)

    vals, idxs = [], []
    cb = lax.broadcasted_iota(jnp.int32, (NCH, 1, QB), 0).astype(jnp.float32) * CH
    for _ in range(ROUNDS):
        mc = jnp.min(d3, axis=1, keepdims=True)
        pick = jnp.where(d3 == mc, iwf, fbig)
        lst = jnp.min(pick, axis=1, keepdims=True)
        d3 = jnp.where(iwf == lst, jnp.inf, d3)
        vals.append(mc)
        idxs.append(lst + cb)
    poolv = jnp.concatenate(vals, axis=1)
    pooli = jnp.concatenate(idxs, axis=1)

    cols = []
    pm = None
    for _ in range(KNN):
        pm = jnp.min(poolv, axis=(0, 1), keepdims=True)
        psel = jnp.where(poolv == pm, pooli, fbig)
        pidx = jnp.min(psel, axis=(0, 1), keepdims=True)
        cols.append(pidx.reshape(1, QB))
        poolv = jnp.where(psel == pidx, jnp.inf, poolv)
    nbrT = jnp.concatenate(cols, axis=0).astype(jnp.int32) + b * n
    nbrT_ref[0] = nbrT

    rem = jnp.min(d3, axis=(0, 1), keepdims=True)
    bad = jnp.any(rem <= pm)

    @pl.when(bad)
    def _fallback():
        dd = build_d3()
        gf = iwf + cb
        cols2 = []
        for _ in range(KNN):
            m = jnp.min(dd, axis=(0, 1), keepdims=True)
            pick = jnp.where(dd == m, gf, fbig)
            idxf = jnp.min(pick, axis=(0, 1), keepdims=True)
            cols2.append(idxf.reshape(1, QB))
            dd = jnp.where(pick == idxf, jnp.inf, dd)
        nbrT_ref[0] = jnp.concatenate(cols2, axis=0).astype(jnp.int32) + b * n

    f = feat_ref[...]
    x = _mm(f, Wenc_ref[...]) + benc_ref[...]
    q = _mm(x, Wq_ref[...]) + bq_ref[...]
    k = _mm(x, Wk_ref[...]) + bk_ref[...]
    v = _mm(x, Wv_ref[...]) + bv_ref[...]
    q_ref[...] = q
    qi = pl.program_id(1)
    pblk = posf_ref[pl.ds(qi * QB, QB), :]
    pad = jnp.zeros((QB, DTAB - 64 - 3), jnp.float32)
    tab_ref[...] = jnp.concatenate([k, v, pblk, pad], axis=1)


def _attn_body(q_ref, g_ref, pos_ref, feat_ref,
               Wp1_ref, bp1_ref, Wp2_ref, bp2_ref,
               Wa1_ref, ba1_ref, Wa2_ref, ba2_ref,
               Wdec_ref, bdec_ref, out_ref):
    C = q_ref.shape[1]
    g = g_ref[...]
    g3 = g.reshape(QC, KNN, DTAB)
    k_n = g3[:, :, 0:C]
    v_n = g3[:, :, C:2 * C]
    pos = pos_ref[...]

    rx = pos[:, None, 0:1] - g3[:, :, 2 * C:2 * C + 1]
    ry = pos[:, None, 1:2] - g3[:, :, 2 * C + 1:2 * C + 2]
    rz = pos[:, None, 2:3] - g3[:, :, 2 * C + 2:2 * C + 3]

    r = lambda t: t.astype(jnp.bfloat16).astype(jnp.float32)
    w0 = r(Wp1_ref[0:1, :])
    w1 = r(Wp1_ref[1:2, :])
    w2 = r(Wp1_ref[2:3, :])
    h = (r(rx) * w0 + r(ry) * w1) + r(rz) * w2 + bp1_ref[...]
    h = jnp.maximum(h, 0.0)
    h2 = h.reshape(QC * KNN, C)
    delta = _mm(h2, Wp2_ref[...]) + bp2_ref[...]
    delta3 = delta.reshape(QC, KNN, C)

    q = q_ref[...]
    e3 = (q[:, None, :] - k_n) + delta3
    e2 = e3.reshape(QC * KNN, C)
    a = jnp.maximum(_mm(e2, Wa1_ref[...]) + ba1_ref[...], 0.0)
    gamma = _mm(a, Wa2_ref[...]) + ba2_ref[...]
    g3m = gamma.reshape(QC, KNN, C)

    mx = jnp.max(g3m, axis=1, keepdims=True)
    ex = jnp.exp(g3m - mx)
    sm = jnp.sum(ex, axis=1, keepdims=True)
    alpha = ex / sm
    out = jnp.sum(alpha * (v_n + delta3), axis=1)

    dec = _mm(out, Wdec_ref[...]) + bdec_ref[...]
    out_ref[...] = feat_ref[...] + dec


def _sc_gather(table, idx_flat):
    rows = idx_flat.shape[0]
    info = plsc.get_sparse_core_info()
    nw = info.num_cores * info.num_subcores
    per_w = rows // nw
    chunk = 128
    n_iter = per_w // chunk
    mesh = plsc.VectorSubcoreMesh(core_axis_name="c", subcore_axis_name="s")

    @functools.partial(
        pl.kernel, mesh=mesh,
        out_type=jax.ShapeDtypeStruct((rows, DTAB), jnp.float32),
        scratch_types=[
            pltpu.VMEM((chunk,), jnp.int32),
            pltpu.VMEM((chunk, DTAB), jnp.float32),
            pltpu.SemaphoreType.DMA,
        ],
    )
    def gather_k(tab_hbm, idx_hbm, out_hbm, idx_v, rows_v, sem):
        wid = lax.axis_index("s") * info.num_cores + lax.axis_index("c")
        base = wid * per_w

        def body(i, carry):
            off = base + i * chunk
            pltpu.sync_copy(idx_hbm.at[pl.ds(off, chunk)], idx_v)
            pltpu.async_copy(tab_hbm.at[idx_v], rows_v, sem).wait()
            pltpu.sync_copy(rows_v, out_hbm.at[pl.ds(off, chunk)])
            return carry

        lax.fori_loop(0, n_iter, body, 0)

    return gather_k(table, idx_flat)


def kernel(features, positions, batch, W_enc, b_enc, Wq, bq, Wk, bk, Wv, bv,
           Wp1, bp1, Wp2, bp2, Wa1, ba1, Wa2, ba2, W_dec, b_dec):
    B = features.shape[0]
    h = B // 2
    out0 = _half(features[:h], positions[:h], W_enc, b_enc, Wq, bq, Wk, bk,
                 Wv, bv, Wp1, bp1, Wp2, bp2, Wa1, ba1, Wa2, ba2, W_dec, b_dec)
    out1 = _half(features[h:], positions[h:], W_enc, b_enc, Wq, bq, Wk, bk,
                 Wv, bv, Wp1, bp1, Wp2, bp2, Wa1, ba1, Wa2, ba2, W_dec, b_dec)
    return (jnp.concatenate([out0, out1], axis=0), positions, batch)


def _half(features, positions, W_enc, b_enc, Wq, bq, Wk, bk, Wv, bv,
          Wp1, bp1, Wp2, bp2, Wa1, ba1, Wa2, ba2, W_dec, b_dec):
    B, N, F = features.shape
    C = Wq.shape[0]
    nbq = N // QB

    f2 = features.reshape(B * N, F)
    pos2 = positions.reshape(B * N, 3)
    posT = jnp.transpose(positions, (0, 2, 1))
    b_enc2 = b_enc.reshape(1, C)
    bq2 = bq.reshape(1, C)
    bk2 = bk.reshape(1, C)
    bv2 = bv.reshape(1, C)

    row_spec = lambda w: pl.BlockSpec((QB, w), lambda b, i: (b * nbq + i, 0))
    full = lambda shape: pl.BlockSpec(shape, lambda b, i: tuple(0 for _ in shape))

    nbrT, q, tab = pl.pallas_call(
        functools.partial(_knn_encode_body, n_points=N),
        grid=(B, nbq),
        in_specs=[
            pl.BlockSpec((1, 3, QB), lambda b, i: (b, 0, i)),
            pl.BlockSpec((N, 3), lambda b, i: (b, 0)),
            row_spec(F),
            full((F, C)), full((1, C)),
            full((C, C)), full((1, C)),
            full((C, C)), full((1, C)),
            full((C, C)), full((1, C)),
        ],
        out_specs=[
            pl.BlockSpec((1, KNN, QB), lambda b, i: (b, 0, i)),
            row_spec(C), row_spec(DTAB),
        ],
        out_shape=[
            jax.ShapeDtypeStruct((B, KNN, N), jnp.int32),
            jax.ShapeDtypeStruct((B * N, C), jnp.float32),
            jax.ShapeDtypeStruct((B * N, DTAB), jnp.float32),
        ],
    )(posT, pos2, f2, W_enc, b_enc2, Wq, bq2, Wk, bk2, Wv, bv2)

    nbr_flat = jnp.transpose(nbrT, (0, 2, 1)).reshape(-1)
    gathered = _sc_gather(tab, nbr_flat)

    nqc = (B * N) // QC
    rs = lambda w: pl.BlockSpec((QC, w), lambda i: (i, 0))
    fullc = lambda shape: pl.BlockSpec(shape, lambda i: tuple(0 for _ in shape))
    featout = pl.pallas_call(
        _attn_body,
        grid=(nqc,),
        in_specs=[
            rs(C),
            pl.BlockSpec((QC * KNN, DTAB), lambda i: (i, 0)),
            rs(3),
            rs(F),
            fullc((3, C)), fullc((1, C)),
            fullc((C, C)), fullc((1, C)),
            fullc((C, C)), fullc((1, C)),
            fullc((C, C)), fullc((1, C)),
            fullc((C, F)), fullc((1, F)),
        ],
        out_specs=rs(F),
        out_shape=jax.ShapeDtypeStruct((B * N, F), jnp.float32),
    )(q, gathered, pos2, f2,
      Wp1, bp1.reshape(1, C), Wp2, bp2.reshape(1, C),
      Wa1, ba1.reshape(1, C), Wa2, ba2.reshape(1, C),
      W_dec, b_dec.reshape(1, F))

    return featout.reshape(B, N, F)


# speedup vs baseline: 27.8232x; 1.0533x over previous
import functools

import jax
import jax.numpy as jnp
from jax import lax
from jax.experimental import pallas as pl
from jax.experimental.pallas import tpu as pltpu
from jax.experimental.pallas import tpu_sc as plsc

KNN = 16
QB = 256
QC = 256
DTAB = 128


def _mm(a, w):
    return jnp.dot(a.astype(jnp.bfloat16), w.astype(jnp.bfloat16),
                   preferred_element_type=jnp.float32)


NCH = 32
CH = 128
ROUNDS = 6


def _knn_encode_body(posq_ref, posf_ref, feat_ref,
                     Wenc_ref, benc_ref, Wq_ref, bq_ref, Wk_ref, bk_ref,
                     Wv_ref, bv_ref,
                     nbrT_ref, q_ref, tab_ref, *, n_points):
    b = pl.program_id(0)
    n = n_points

    r = lambda t: t.astype(jnp.bfloat16).astype(jnp.float32)
    pq = posq_ref[0]
    xq = pq[0:1, :]
    yq = pq[1:2, :]
    zq = pq[2:3, :]
    sqq = (xq * xq + yq * yq) + zq * zq
    pf = posf_ref[...]
    xa = pf[:, 0:1]
    ya = pf[:, 1:2]
    za = pf[:, 2:3]
    sqa = (xa * xa + ya * ya) + za * za

    def build_d3():
        dot = (r(xa) * r(xq) + r(ya) * r(yq)) + r(za) * r(zq)
        dT = (sqq + sqa) - 2.0 * dot
        return dT.reshape(NCH, CH, QB)

    d3 = build_d3()
    iwf = lax.broadcasted_iota(jnp.int32, (NCH, CH, QB), 1).astype(jnp.float32)
    fbig = jnp.float32(1e9)

    vals, idxs = [], []
    cb = lax.broadcasted_iota(jnp.int32, (NCH, QB), 0).astype(jnp.float32) * CH
    for _ in range(ROUNDS):
        mc = jnp.min(d3, axis=1, keepdims=True)
        pick = jnp.where(d3 == mc, iwf, fbig)
        lst = jnp.min(pick, axis=1, keepdims=True)
        d3 = jnp.where(iwf == lst, jnp.inf, d3)
        vals.append(mc.reshape(NCH, QB))
        idxs.append(lst.reshape(NCH, QB) + cb)
    poolv = jnp.concatenate(vals, axis=0)
    pooli = jnp.concatenate(idxs, axis=0)

    cols = []
    pm = None
    for _ in range(KNN):
        pm = jnp.min(poolv, axis=0, keepdims=True)
        psel = jnp.where(poolv == pm, pooli, fbig)
        pidx = jnp.min(psel, axis=0, keepdims=True)
        cols.append(pidx)
        poolv = jnp.where(psel == pidx, jnp.inf, poolv)
    nbrT = jnp.concatenate(cols, axis=0).astype(jnp.int32) + b * n
    nbrT_ref[0] = nbrT

    rem = jnp.min(jnp.min(d3, axis=1).reshape(NCH, QB), axis=0, keepdims=True)
    bad = jnp.any(rem <= pm)

    @pl.when(bad)
    def _fallback():
        dd = build_d3()
        gf = iwf + cb.reshape(NCH, 1, QB)
        cols2 = []
        for _ in range(KNN):
            m2 = jnp.min(jnp.min(dd, axis=1).reshape(NCH, QB),
                         axis=0, keepdims=True)
            m = m2.reshape(1, 1, QB)
            pick = jnp.where(dd == m, gf, fbig)
            i2 = jnp.min(jnp.min(pick, axis=1).reshape(NCH, QB),
                         axis=0, keepdims=True)
            cols2.append(i2)
            dd = jnp.where(pick == i2.reshape(1, 1, QB), jnp.inf, dd)
        nbrT_ref[0] = jnp.concatenate(cols2, axis=0).astype(jnp.int32) + b * n

    f = feat_ref[...]
    x = _mm(f, Wenc_ref[...]) + benc_ref[...]
    q = _mm(x, Wq_ref[...]) + bq_ref[...]
    k = _mm(x, Wk_ref[...]) + bk_ref[...]
    v = _mm(x, Wv_ref[...]) + bv_ref[...]
    q_ref[...] = q
    qi = pl.program_id(1)
    pblk = posf_ref[pl.ds(qi * QB, QB), :]
    pad = jnp.zeros((QB, DTAB - 64 - 3), jnp.float32)
    tab_ref[...] = jnp.concatenate([k, v, pblk, pad], axis=1)


def _attn_body(q_ref, g_ref, pos_ref, feat_ref,
               Wp1_ref, bp1_ref, Wp2_ref, bp2_ref,
               Wa1_ref, ba1_ref, Wa2_ref, ba2_ref,
               Wdec_ref, bdec_ref, out_ref):
    C = q_ref.shape[1]
    g = g_ref[...]
    g3 = g.reshape(QC, KNN, DTAB)
    k_n = g3[:, :, 0:C]
    v_n = g3[:, :, C:2 * C]
    pos = pos_ref[...]

    rx = pos[:, None, 0:1] - g3[:, :, 2 * C:2 * C + 1]
    ry = pos[:, None, 1:2] - g3[:, :, 2 * C + 1:2 * C + 2]
    rz = pos[:, None, 2:3] - g3[:, :, 2 * C + 2:2 * C + 3]

    r = lambda t: t.astype(jnp.bfloat16).astype(jnp.float32)
    w0 = r(Wp1_ref[0:1, :])
    w1 = r(Wp1_ref[1:2, :])
    w2 = r(Wp1_ref[2:3, :])
    h = (r(rx) * w0 + r(ry) * w1) + r(rz) * w2 + bp1_ref[...]
    h = jnp.maximum(h, 0.0)
    h2 = h.reshape(QC * KNN, C)
    delta = _mm(h2, Wp2_ref[...]) + bp2_ref[...]
    delta3 = delta.reshape(QC, KNN, C)

    q = q_ref[...]
    e3 = (q[:, None, :] - k_n) + delta3
    e2 = e3.reshape(QC * KNN, C)
    a = jnp.maximum(_mm(e2, Wa1_ref[...]) + ba1_ref[...], 0.0)
    gamma = _mm(a, Wa2_ref[...]) + ba2_ref[...]
    g3m = gamma.reshape(QC, KNN, C)

    mx = jnp.max(g3m, axis=1, keepdims=True)
    ex = jnp.exp(g3m - mx)
    sm = jnp.sum(ex, axis=1, keepdims=True)
    alpha = ex / sm
    out = jnp.sum(alpha * (v_n + delta3), axis=1)

    dec = _mm(out, Wdec_ref[...]) + bdec_ref[...]
    out_ref[...] = feat_ref[...] + dec


def _sc_gather(table, idx_flat):
    rows = idx_flat.shape[0]
    info = plsc.get_sparse_core_info()
    nw = info.num_cores * info.num_subcores
    per_w = rows // nw
    chunk = 128
    n_iter = per_w // chunk
    mesh = plsc.VectorSubcoreMesh(core_axis_name="c", subcore_axis_name="s")

    @functools.partial(
        pl.kernel, mesh=mesh,
        out_type=jax.ShapeDtypeStruct((rows, DTAB), jnp.float32),
        scratch_types=[
            pltpu.VMEM((chunk,), jnp.int32),
            pltpu.VMEM((chunk, DTAB), jnp.float32),
            pltpu.SemaphoreType.DMA,
        ],
    )
    def gather_k(tab_hbm, idx_hbm, out_hbm, idx_v, rows_v, sem):
        wid = lax.axis_index("s") * info.num_cores + lax.axis_index("c")
        base = wid * per_w

        def body(i, carry):
            off = base + i * chunk
            pltpu.sync_copy(idx_hbm.at[pl.ds(off, chunk)], idx_v)
            pltpu.async_copy(tab_hbm.at[idx_v], rows_v, sem).wait()
            pltpu.sync_copy(rows_v, out_hbm.at[pl.ds(off, chunk)])
            return carry

        lax.fori_loop(0, n_iter, body, 0)

    return gather_k(table, idx_flat)


def kernel(features, positions, batch, W_enc, b_enc, Wq, bq, Wk, bk, Wv, bv,
           Wp1, bp1, Wp2, bp2, Wa1, ba1, Wa2, ba2, W_dec, b_dec):
    B = features.shape[0]
    h = B // 2
    out0 = _half(features[:h], positions[:h], W_enc, b_enc, Wq, bq, Wk, bk,
                 Wv, bv, Wp1, bp1, Wp2, bp2, Wa1, ba1, Wa2, ba2, W_dec, b_dec)
    out1 = _half(features[h:], positions[h:], W_enc, b_enc, Wq, bq, Wk, bk,
                 Wv, bv, Wp1, bp1, Wp2, bp2, Wa1, ba1, Wa2, ba2, W_dec, b_dec)
    return (jnp.concatenate([out0, out1], axis=0), positions, batch)


def _half(features, positions, W_enc, b_enc, Wq, bq, Wk, bk, Wv, bv,
          Wp1, bp1, Wp2, bp2, Wa1, ba1, Wa2, ba2, W_dec, b_dec):
    B, N, F = features.shape
    C = Wq.shape[0]
    nbq = N // QB

    f2 = features.reshape(B * N, F)
    pos2 = positions.reshape(B * N, 3)
    posT = jnp.transpose(positions, (0, 2, 1))
    b_enc2 = b_enc.reshape(1, C)
    bq2 = bq.reshape(1, C)
    bk2 = bk.reshape(1, C)
    bv2 = bv.reshape(1, C)

    row_spec = lambda w: pl.BlockSpec((QB, w), lambda b, i: (b * nbq + i, 0))
    full = lambda shape: pl.BlockSpec(shape, lambda b, i: tuple(0 for _ in shape))

    nbrT, q, tab = pl.pallas_call(
        functools.partial(_knn_encode_body, n_points=N),
        grid=(B, nbq),
        in_specs=[
            pl.BlockSpec((1, 3, QB), lambda b, i: (b, 0, i)),
            pl.BlockSpec((N, 3), lambda b, i: (b, 0)),
            row_spec(F),
            full((F, C)), full((1, C)),
            full((C, C)), full((1, C)),
            full((C, C)), full((1, C)),
            full((C, C)), full((1, C)),
        ],
        out_specs=[
            pl.BlockSpec((1, KNN, QB), lambda b, i: (b, 0, i)),
            row_spec(C), row_spec(DTAB),
        ],
        out_shape=[
            jax.ShapeDtypeStruct((B, KNN, N), jnp.int32),
            jax.ShapeDtypeStruct((B * N, C), jnp.float32),
            jax.ShapeDtypeStruct((B * N, DTAB), jnp.float32),
        ],
    )(posT, pos2, f2, W_enc, b_enc2, Wq, bq2, Wk, bk2, Wv, bv2)

    nbr_flat = jnp.transpose(nbrT, (0, 2, 1)).reshape(-1)
    gathered = _sc_gather(tab, nbr_flat)

    nqc = (B * N) // QC
    rs = lambda w: pl.BlockSpec((QC, w), lambda i: (i, 0))
    fullc = lambda shape: pl.BlockSpec(shape, lambda i: tuple(0 for _ in shape))
    featout = pl.pallas_call(
        _attn_body,
        grid=(nqc,),
        in_specs=[
            rs(C),
            pl.BlockSpec((QC * KNN, DTAB), lambda i: (i, 0)),
            rs(3),
            rs(F),
            fullc((3, C)), fullc((1, C)),
            fullc((C, C)), fullc((1, C)),
            fullc((C, C)), fullc((1, C)),
            fullc((C, C)), fullc((1, C)),
            fullc((C, F)), fullc((1, F)),
        ],
        out_specs=rs(F),
        out_shape=jax.ShapeDtypeStruct((B * N, F), jnp.float32),
    )(q, gathered, pos2, f2,
      Wp1, bp1.reshape(1, C), Wp2, bp2.reshape(1, C),
      Wa1, ba1.reshape(1, C), Wa2, ba2.reshape(1, C),
      W_dec, b_dec.reshape(1, F))

    return featout.reshape(B, N, F)


# speedup vs baseline: 29.7063x; 1.0677x over previous
import functools

import jax
import jax.numpy as jnp
from jax import lax
from jax.experimental import pallas as pl
from jax.experimental.pallas import tpu as pltpu
from jax.experimental.pallas import tpu_sc as plsc

KNN = 16
QB = 256
QC = 256
DTAB = 128


def _mm(a, w):
    return jnp.dot(a.astype(jnp.bfloat16), w.astype(jnp.bfloat16),
                   preferred_element_type=jnp.float32)


NCH = 32
CH = 128
ROUNDS = 6


def _knn_encode_body(posq_ref, posf_ref, feat_ref,
                     Wenc_ref, benc_ref, Wq_ref, bq_ref, Wk_ref, bk_ref,
                     Wv_ref, bv_ref,
                     nbrT_ref, q_ref, tab_ref, *, n_points):
    b = pl.program_id(0)
    n = n_points

    r = lambda t: t.astype(jnp.bfloat16).astype(jnp.float32)
    pq = posq_ref[0]
    xq = pq[0:1, :]
    yq = pq[1:2, :]
    zq = pq[2:3, :]
    sqq = (xq * xq + yq * yq) + zq * zq
    pf = posf_ref[...]
    xa = pf[:, 0:1]
    ya = pf[:, 1:2]
    za = pf[:, 2:3]
    sqa = (xa * xa + ya * ya) + za * za

    def build_d3():
        dot = (r(xa) * r(xq) + r(ya) * r(yq)) + r(za) * r(zq)
        dT = (sqq + sqa) - 2.0 * dot
        return dT.reshape(NCH, CH, QB)

    d3 = build_d3()
    iwf = lax.broadcasted_iota(jnp.int32, (NCH, CH, QB), 1).astype(jnp.float32)
    fbig = jnp.float32(1e9)

    def chunk_argmin(v):
        ix = iwf
        w = CH
        while w > 1:
            h = w // 2
            t = v[:, h:w] < v[:, :h]
            v = jnp.where(t, v[:, h:w], v[:, :h])
            ix = jnp.where(t, ix[:, h:w], ix[:, :h])
            w = h
        return v, ix

    vals, idxs = [], []
    cb = lax.broadcasted_iota(jnp.int32, (NCH, QB), 0).astype(jnp.float32) * CH
    for _ in range(ROUNDS):
        mc, lst = chunk_argmin(d3)
        d3 = jnp.where(iwf == lst, jnp.inf, d3)
        vals.append(mc.reshape(NCH, QB))
        idxs.append(lst.reshape(NCH, QB) + cb)
    poolv = jnp.concatenate(vals, axis=0)
    pooli = jnp.concatenate(idxs, axis=0)

    cols = []
    pm = None
    for _ in range(KNN):
        pm = jnp.min(poolv, axis=0, keepdims=True)
        psel = jnp.where(poolv == pm, pooli, fbig)
        pidx = jnp.min(psel, axis=0, keepdims=True)
        cols.append(pidx)
        poolv = jnp.where(psel == pidx, jnp.inf, poolv)
    nbrT = jnp.concatenate(cols, axis=0).astype(jnp.int32) + b * n
    nbrT_ref[0] = nbrT

    def chunk_min(v):
        w = CH
        while w > 1:
            h = w // 2
            v = jnp.minimum(v[:, h:w], v[:, :h])
            w = h
        return v

    rem = jnp.min(chunk_min(d3).reshape(NCH, QB), axis=0, keepdims=True)
    bad = jnp.any(rem <= pm)

    @pl.when(bad)
    def _fallback():
        dd = build_d3()
        gf = iwf + cb.reshape(NCH, 1, QB)
        cols2 = []
        for _ in range(KNN):
            m2 = jnp.min(jnp.min(dd, axis=1).reshape(NCH, QB),
                         axis=0, keepdims=True)
            m = m2.reshape(1, 1, QB)
            pick = jnp.where(dd == m, gf, fbig)
            i2 = jnp.min(jnp.min(pick, axis=1).reshape(NCH, QB),
                         axis=0, keepdims=True)
            cols2.append(i2)
            dd = jnp.where(pick == i2.reshape(1, 1, QB), jnp.inf, dd)
        nbrT_ref[0] = jnp.concatenate(cols2, axis=0).astype(jnp.int32) + b * n

    f = feat_ref[...]
    x = _mm(f, Wenc_ref[...]) + benc_ref[...]
    q = _mm(x, Wq_ref[...]) + bq_ref[...]
    k = _mm(x, Wk_ref[...]) + bk_ref[...]
    v = _mm(x, Wv_ref[...]) + bv_ref[...]
    q_ref[...] = q
    qi = pl.program_id(1)
    pblk = posf_ref[pl.ds(qi * QB, QB), :]
    pad = jnp.zeros((QB, DTAB - 64 - 3), jnp.float32)
    tab_ref[...] = jnp.concatenate([k, v, pblk, pad], axis=1)


def _attn_body(q_ref, g_ref, pos_ref, feat_ref,
               Wp1_ref, bp1_ref, Wp2_ref, bp2_ref,
               Wa1_ref, ba1_ref, Wa2_ref, ba2_ref,
               Wdec_ref, bdec_ref, out_ref):
    C = q_ref.shape[1]
    g = g_ref[...]
    g3 = g.reshape(QC, KNN, DTAB)
    k_n = g3[:, :, 0:C]
    v_n = g3[:, :, C:2 * C]
    pos = pos_ref[...]

    rx = pos[:, None, 0:1] - g3[:, :, 2 * C:2 * C + 1]
    ry = pos[:, None, 1:2] - g3[:, :, 2 * C + 1:2 * C + 2]
    rz = pos[:, None, 2:3] - g3[:, :, 2 * C + 2:2 * C + 3]

    r = lambda t: t.astype(jnp.bfloat16).astype(jnp.float32)
    w0 = r(Wp1_ref[0:1, :])
    w1 = r(Wp1_ref[1:2, :])
    w2 = r(Wp1_ref[2:3, :])
    h = (r(rx) * w0 + r(ry) * w1) + r(rz) * w2 + bp1_ref[...]
    h = jnp.maximum(h, 0.0)
    h2 = h.reshape(QC * KNN, C)
    delta = _mm(h2, Wp2_ref[...]) + bp2_ref[...]
    delta3 = delta.reshape(QC, KNN, C)

    q = q_ref[...]
    e3 = (q[:, None, :] - k_n) + delta3
    e2 = e3.reshape(QC * KNN, C)
    a = jnp.maximum(_mm(e2, Wa1_ref[...]) + ba1_ref[...], 0.0)
    gamma = _mm(a, Wa2_ref[...]) + ba2_ref[...]
    g3m = gamma.reshape(QC, KNN, C)

    mx = jnp.max(g3m, axis=1, keepdims=True)
    ex = jnp.exp(g3m - mx)
    sm = jnp.sum(ex, axis=1, keepdims=True)
    alpha = ex / sm
    out = jnp.sum(alpha * (v_n + delta3), axis=1)

    dec = _mm(out, Wdec_ref[...]) + bdec_ref[...]
    out_ref[...] = feat_ref[...] + dec


def _sc_gather(table, idx_flat):
    rows = idx_flat.shape[0]
    info = plsc.get_sparse_core_info()
    nw = info.num_cores * info.num_subcores
    per_w = rows // nw
    chunk = 128
    n_iter = per_w // chunk
    mesh = plsc.VectorSubcoreMesh(core_axis_name="c", subcore_axis_name="s")

    @functools.partial(
        pl.kernel, mesh=mesh,
        out_type=jax.ShapeDtypeStruct((rows, DTAB), jnp.float32),
        scratch_types=[
            pltpu.VMEM((chunk,), jnp.int32),
            pltpu.VMEM((chunk, DTAB), jnp.float32),
            pltpu.SemaphoreType.DMA,
        ],
    )
    def gather_k(tab_hbm, idx_hbm, out_hbm, idx_v, rows_v, sem):
        wid = lax.axis_index("s") * info.num_cores + lax.axis_index("c")
        base = wid * per_w

        def body(i, carry):
            off = base + i * chunk
            pltpu.sync_copy(idx_hbm.at[pl.ds(off, chunk)], idx_v)
            pltpu.async_copy(tab_hbm.at[idx_v], rows_v, sem).wait()
            pltpu.sync_copy(rows_v, out_hbm.at[pl.ds(off, chunk)])
            return carry

        lax.fori_loop(0, n_iter, body, 0)

    return gather_k(table, idx_flat)


def kernel(features, positions, batch, W_enc, b_enc, Wq, bq, Wk, bk, Wv, bv,
           Wp1, bp1, Wp2, bp2, Wa1, ba1, Wa2, ba2, W_dec, b_dec):
    B = features.shape[0]
    h = B // 2
    out0 = _half(features[:h], positions[:h], W_enc, b_enc, Wq, bq, Wk, bk,
                 Wv, bv, Wp1, bp1, Wp2, bp2, Wa1, ba1, Wa2, ba2, W_dec, b_dec)
    out1 = _half(features[h:], positions[h:], W_enc, b_enc, Wq, bq, Wk, bk,
                 Wv, bv, Wp1, bp1, Wp2, bp2, Wa1, ba1, Wa2, ba2, W_dec, b_dec)
    return (jnp.concatenate([out0, out1], axis=0), positions, batch)


def _half(features, positions, W_enc, b_enc, Wq, bq, Wk, bk, Wv, bv,
          Wp1, bp1, Wp2, bp2, Wa1, ba1, Wa2, ba2, W_dec, b_dec):
    B, N, F = features.shape
    C = Wq.shape[0]
    nbq = N // QB

    f2 = features.reshape(B * N, F)
    pos2 = positions.reshape(B * N, 3)
    posT = jnp.transpose(positions, (0, 2, 1))
    b_enc2 = b_enc.reshape(1, C)
    bq2 = bq.reshape(1, C)
    bk2 = bk.reshape(1, C)
    bv2 = bv.reshape(1, C)

    row_spec = lambda w: pl.BlockSpec((QB, w), lambda b, i: (b * nbq + i, 0))
    full = lambda shape: pl.BlockSpec(shape, lambda b, i: tuple(0 for _ in shape))

    nbrT, q, tab = pl.pallas_call(
        functools.partial(_knn_encode_body, n_points=N),
        grid=(B, nbq),
        in_specs=[
            pl.BlockSpec((1, 3, QB), lambda b, i: (b, 0, i)),
            pl.BlockSpec((N, 3), lambda b, i: (b, 0)),
            row_spec(F),
            full((F, C)), full((1, C)),
            full((C, C)), full((1, C)),
            full((C, C)), full((1, C)),
            full((C, C)), full((1, C)),
        ],
        out_specs=[
            pl.BlockSpec((1, KNN, QB), lambda b, i: (b, 0, i)),
            row_spec(C), row_spec(DTAB),
        ],
        out_shape=[
            jax.ShapeDtypeStruct((B, KNN, N), jnp.int32),
            jax.ShapeDtypeStruct((B * N, C), jnp.float32),
            jax.ShapeDtypeStruct((B * N, DTAB), jnp.float32),
        ],
    )(posT, pos2, f2, W_enc, b_enc2, Wq, bq2, Wk, bk2, Wv, bv2)

    nbr_flat = jnp.transpose(nbrT, (0, 2, 1)).reshape(-1)
    gathered = _sc_gather(tab, nbr_flat)

    nqc = (B * N) // QC
    rs = lambda w: pl.BlockSpec((QC, w), lambda i: (i, 0))
    fullc = lambda shape: pl.BlockSpec(shape, lambda i: tuple(0 for _ in shape))
    featout = pl.pallas_call(
        _attn_body,
        grid=(nqc,),
        in_specs=[
            rs(C),
            pl.BlockSpec((QC * KNN, DTAB), lambda i: (i, 0)),
            rs(3),
            rs(F),
            fullc((3, C)), fullc((1, C)),
            fullc((C, C)), fullc((1, C)),
            fullc((C, C)), fullc((1, C)),
            fullc((C, C)), fullc((1, C)),
            fullc((C, F)), fullc((1, F)),
        ],
        out_specs=rs(F),
        out_shape=jax.ShapeDtypeStruct((B * N, F), jnp.float32),
    )(q, gathered, pos2, f2,
      Wp1, bp1.reshape(1, C), Wp2, bp2.reshape(1, C),
      Wa1, ba1.reshape(1, C), Wa2, ba2.reshape(1, C),
      W_dec, b_dec.reshape(1, F))

    return featout.reshape(B, N, F)
